# Initial kernel scaffold; baseline (speedup 1.0000x reference)
import jax
import jax.numpy as jnp
from jax.experimental import pallas as pl


def kernel(lang_code, lang_code_table):
    raise NotImplementedError("write your pallas kernel here")


# speedup vs baseline: 6.2669x; 6.2669x over previous
import functools

import jax
import jax.numpy as jnp
from jax import lax
from jax.experimental import pallas as pl
from jax.experimental.pallas import tpu as pltpu
from jax.experimental.pallas import tpu_sc as plsc

NC, NS = 2, 16
NW = NC * NS
N = 16384 * 200
D = 32
PER_W = N // NW
CHUNK = 1600
NBUF = 2
STEPS = PER_W // CHUNK
GROUPS = STEPS // NBUF

_mesh = plsc.VectorSubcoreMesh(
    core_axis_name="c", subcore_axis_name="s", num_cores=NC, num_subcores=NS
)


@functools.partial(
    pl.kernel,
    out_type=jax.ShapeDtypeStruct((N, D), jnp.float32),
    mesh=_mesh,
    scratch_types=[
        pltpu.VMEM_SHARED((5, D), jnp.float32),
        pltpu.VMEM((NBUF, CHUNK), jnp.int32),
        pltpu.VMEM((NBUF, CHUNK, D), jnp.float32),
        pltpu.SemaphoreType.DMA((NBUF,)),
        pltpu.SemaphoreType.DMA((NBUF,)),
        pltpu.SemaphoreType.DMA((NBUF,)),
    ],
    compiler_params=pltpu.CompilerParams(use_tc_tiling_on_sc=False),
)
def _lookup(idx_hbm, table_hbm, out_hbm, table_v, idx_buf, row_buf,
            isems, gsems, osems):
    sid = lax.axis_index("s")
    wid = sid * NC + lax.axis_index("c")
    base = wid * PER_W

    @pl.when(sid == 0)
    def _():
        pltpu.sync_copy(table_hbm, table_v)

    plsc.subcore_barrier()

    for b in range(NBUF):
        pltpu.async_copy(
            idx_hbm.at[pl.ds(base + b * CHUNK, CHUNK)], idx_buf.at[b],
            isems.at[b])

    def group(g, carry):
        i0 = g * NBUF
        for b in range(NBUF):
            off = base + (i0 + b) * CHUNK
            pltpu.make_async_copy(
                idx_hbm.at[pl.ds(off, CHUNK)], idx_buf.at[b], isems.at[b]
            ).wait()

            @pl.when(g > 0)
            def _():
                pltpu.make_async_copy(
                    row_buf.at[b], out_hbm.at[pl.ds(off, CHUNK)], osems.at[b]
                ).wait()

            pltpu.async_copy(table_v.at[idx_buf.at[b]], row_buf.at[b],
                             gsems.at[b])

        for b in range(NBUF):
            off = base + (i0 + b) * CHUNK
            pltpu.make_async_copy(
                table_v.at[idx_buf.at[b]], row_buf.at[b], gsems.at[b]
            ).wait()
            pltpu.async_copy(row_buf.at[b], out_hbm.at[pl.ds(off, CHUNK)],
                             osems.at[b])

            @pl.when(g < GROUPS - 1)
            def _():
                pltpu.async_copy(
                    idx_hbm.at[pl.ds(off + NBUF * CHUNK, CHUNK)],
                    idx_buf.at[b], isems.at[b])

        return carry

    lax.fori_loop(0, GROUPS, group, 0)

    for b in range(NBUF):
        pltpu.make_async_copy(
            row_buf.at[b], out_hbm.at[pl.ds(base, CHUNK)], osems.at[b]
        ).wait()


def kernel(lang_code, lang_code_table):
    idx = lang_code.astype(jnp.int32).reshape(N)
    out = _lookup(idx, lang_code_table)
    return out.reshape(16384, 200, D)
